# baseline (speedup 1.0000x reference)
import functools

import jax
import jax.numpy as jnp
from jax import lax
from jax.experimental import pallas as pl
from jax.experimental.pallas import tpu as pltpu
from jax.experimental.pallas import tpu_sc as plsc

SEQ = 2048
K = 16
NE = SEQ * K
TW = 128
CH = 128
SQRT_EPS = 1e-3
DIST_EPS = 0.01

NCORES = 2
NSUB = 16
NW = NCORES * NSUB


def _table_body(xp_ref, c_ref, t_ref):
    n = c_ref.shape[0]
    m = (c_ref[...] > 0).astype(jnp.float32)
    z1 = jnp.zeros((1, 1), jnp.float32)
    m3 = jnp.concatenate([z1, m[:-1]], axis=0) * m * \
        jnp.concatenate([m[1:], z1], axis=0)
    z4 = jnp.zeros((1, 4), jnp.float32)
    planes = []
    for d in range(3):
        xd = xp_ref[d]
        left = jnp.concatenate([z4, xd[:-1]], axis=0)
        right = jnp.concatenate([xd[1:], z4], axis=0)
        planes.append(jnp.concatenate([left, xd, right], axis=1))
    pad = jnp.zeros((n, TW - 37), jnp.float32)
    t_ref[...] = jnp.concatenate(planes + [m3, pad], axis=1)


def _build_table(xp, c2):
    return pl.pallas_call(
        _table_body,
        out_shape=jax.ShapeDtypeStruct((SEQ, TW), jnp.float32),
    )(xp, c2)


def _sc_gather(table, idx2d):
    bpw = NE // NW
    nch = bpw // CH
    mesh = plsc.VectorSubcoreMesh(core_axis_name="c", subcore_axis_name="s")

    @functools.partial(
        pl.kernel,
        mesh=mesh,
        out_type=jax.ShapeDtypeStruct((NE, TW), jnp.float32),
        scratch_types=[
            pltpu.VMEM((nch, CH), jnp.int32),
            pltpu.VMEM((4 * CH, TW), jnp.float32),
            pltpu.SemaphoreType.DMA,
        ],
    )
    def gk(table_hbm, idx_hbm, out_hbm, idx_v, rows_v, sem):
        wid = lax.axis_index("s") * NCORES + lax.axis_index("c")
        base = wid * bpw
        pltpu.sync_copy(idx_hbm.at[pl.ds(wid * nch, nch)], idx_v)
        for r in range(nch // 4):
            copies = []
            for jj in range(4):
                copies.append(pltpu.async_copy(
                    table_hbm.at[idx_v.at[r * 4 + jj]],
                    rows_v.at[pl.ds(jj * CH, CH)],
                    sem,
                ))
            for c in copies:
                c.wait()
            pltpu.sync_copy(
                rows_v, out_hbm.at[pl.ds(base + r * 4 * CH, 4 * CH)])

    return gk(table, idx2d)


BL = 8
EB = BL * K


def _edge_body(t_ref, g_ref, o_ref):
    ti = t_ref[...]
    g = g_ref[...]
    tib = jnp.broadcast_to(ti[:, None, :], (BL, K, TW)).reshape(EB, TW)
    mask = tib[:, 36:37] * g[:, 36:37]
    ssq = jnp.full((EB, 576), SQRT_EPS, jnp.float32)
    for d in range(3):
        qd = jnp.concatenate(
            [tib[:, 12 * d:12 * (d + 1)], g[:, 12 * d:12 * (d + 1)]], axis=1)
        ad = jnp.concatenate(
            [jnp.broadcast_to(qd[:, p:p + 1], (EB, 24)) for p in range(24)],
            axis=1)
        bd = jnp.concatenate([qd] * 24, axis=1)
        diff = ad - bd
        ssq = ssq + diff * diff
    o_ref[...] = jnp.log(jnp.sqrt(ssq) + DIST_EPS) * mask


def _edge_features(table, g):
    return pl.pallas_call(
        _edge_body,
        grid=(SEQ // BL,),
        in_specs=[
            pl.BlockSpec((BL, TW), lambda i: (i, 0)),
            pl.BlockSpec((EB, TW), lambda i: (i, 0)),
        ],
        out_specs=pl.BlockSpec((EB, 576), lambda i: (i, 0)),
        out_shape=jax.ShapeDtypeStruct((NE, 576), jnp.float32),
    )(table, g)


def kernel(X, edge_idx, C):
    b, n = X.shape[0], X.shape[1]
    xp = jnp.transpose(X[0], (2, 0, 1))
    c2 = C[0][:, None].astype(jnp.int32)
    table = _build_table(xp, c2)
    idx2d = edge_idx[0].reshape(NE // CH, CH).astype(jnp.int32)
    g = _sc_gather(table, idx2d)
    out = _edge_features(table, g)
    return out.reshape(b, n, K, 576)


# speedup vs baseline: 2.5902x; 2.5902x over previous
import functools

import jax
import jax.numpy as jnp
from jax import lax
from jax.experimental import pallas as pl
from jax.experimental.pallas import tpu as pltpu
from jax.experimental.pallas import tpu_sc as plsc

SEQ = 2048
K = 16
NE = SEQ * K
TW = 128
CH = 128
SQRT_EPS = 1e-3
DIST_EPS = 0.01

NCORES = 2
NSUB = 16
NW = NCORES * NSUB


def _table_body(xp_ref, c_ref, t_ref):
    n = c_ref.shape[0]
    m = (c_ref[...] > 0).astype(jnp.float32)
    z1 = jnp.zeros((1, 1), jnp.float32)
    m3 = jnp.concatenate([z1, m[:-1]], axis=0) * m * \
        jnp.concatenate([m[1:], z1], axis=0)
    z4 = jnp.zeros((1, 4), jnp.float32)
    planes = []
    for d in range(3):
        xd = xp_ref[d]
        left = jnp.concatenate([z4, xd[:-1]], axis=0)
        right = jnp.concatenate([xd[1:], z4], axis=0)
        planes.append(jnp.concatenate([left, xd, right], axis=1))
    pad = jnp.zeros((n, TW - 37), jnp.float32)
    t_ref[...] = jnp.concatenate(planes + [m3, pad], axis=1)


def _build_table(xp, c2):
    return pl.pallas_call(
        _table_body,
        out_shape=jax.ShapeDtypeStruct((SEQ, TW), jnp.float32),
    )(xp, c2)


def _sc_gather(table, idx2d):
    bpw = NE // NW
    nch = bpw // CH
    mesh = plsc.VectorSubcoreMesh(core_axis_name="c", subcore_axis_name="s")

    @functools.partial(
        pl.kernel,
        mesh=mesh,
        out_type=jax.ShapeDtypeStruct((NE, TW), jnp.float32),
        scratch_types=[
            pltpu.VMEM((nch, CH), jnp.int32),
            pltpu.VMEM((4 * CH, TW), jnp.float32),
            pltpu.SemaphoreType.DMA,
        ],
    )
    def gk(table_hbm, idx_hbm, out_hbm, idx_v, rows_v, sem):
        wid = lax.axis_index("s") * NCORES + lax.axis_index("c")
        base = wid * bpw
        pltpu.sync_copy(idx_hbm.at[pl.ds(wid * nch, nch)], idx_v)
        for r in range(nch // 4):
            copies = []
            for jj in range(4):
                copies.append(pltpu.async_copy(
                    table_hbm.at[idx_v.at[r * 4 + jj]],
                    rows_v.at[pl.ds(jj * CH, CH)],
                    sem,
                ))
            for c in copies:
                c.wait()
            pltpu.sync_copy(
                rows_v, out_hbm.at[pl.ds(base + r * 4 * CH, 4 * CH)])

    return gk(table, idx2d)


BL = 16
EB = BL * K
CW = 640


def _diff_matrix():
    import numpy as np
    md = np.zeros((72, 3 * CW), np.float32)
    for d in range(3):
        for p in range(24):
            for q in range(24):
                col = CW * d + 24 * p + q
                md[24 * d + p, col] += 1.0
                md[24 * d + q, col] -= 1.0
    return jnp.asarray(md)


def _edge_body(t_ref, g_ref, md_ref, o_ref):
    ti = t_ref[...]
    g = g_ref[...]
    tib = jnp.broadcast_to(ti[:, None, :], (BL, K, TW)).reshape(EB, TW)
    mask = tib[:, 36:37] * g[:, 36:37]
    q72 = jnp.concatenate(
        [jnp.concatenate(
            [tib[:, 12 * d:12 * (d + 1)], g[:, 12 * d:12 * (d + 1)]], axis=1)
         for d in range(3)], axis=1)
    diff = jnp.dot(q72, md_ref[...],
                   precision=jax.lax.Precision.HIGHEST)
    sq = diff * diff
    ssq = sq[:, :CW] + sq[:, CW:2 * CW] + sq[:, 2 * CW:] + SQRT_EPS
    feat = jnp.log(jnp.sqrt(ssq) + DIST_EPS) * mask
    o_ref[...] = feat[:, :576]


def _edge_features(table, g):
    return pl.pallas_call(
        _edge_body,
        grid=(SEQ // BL,),
        in_specs=[
            pl.BlockSpec((BL, TW), lambda i: (i, 0)),
            pl.BlockSpec((EB, TW), lambda i: (i, 0)),
            pl.BlockSpec((72, 3 * CW), lambda i: (0, 0)),
        ],
        out_specs=pl.BlockSpec((EB, 576), lambda i: (i, 0)),
        out_shape=jax.ShapeDtypeStruct((NE, 576), jnp.float32),
    )(table, g, _diff_matrix())


def kernel(X, edge_idx, C):
    b, n = X.shape[0], X.shape[1]
    xp = jnp.transpose(X[0], (2, 0, 1))
    c2 = C[0][:, None].astype(jnp.int32)
    table = _build_table(xp, c2)
    idx2d = edge_idx[0].reshape(NE // CH, CH).astype(jnp.int32)
    g = _sc_gather(table, idx2d)
    out = _edge_features(table, g)
    return out.reshape(b, n, K, 576)


# speedup vs baseline: 3.7625x; 1.4526x over previous
import functools

import jax
import jax.numpy as jnp
from jax import lax
from jax.experimental import pallas as pl
from jax.experimental.pallas import tpu as pltpu
from jax.experimental.pallas import tpu_sc as plsc

SEQ = 2048
K = 16
NE = SEQ * K
TW = 128
CH = 128
SQRT_EPS = 1e-3
DIST_EPS = 0.01

NCORES = 2
NSUB = 16
NW = NCORES * NSUB


def _table_body(xp_ref, c_ref, t_ref):
    n = c_ref.shape[0]
    m = (c_ref[...] > 0).astype(jnp.float32)
    z1 = jnp.zeros((1, 1), jnp.float32)
    m3 = jnp.concatenate([z1, m[:-1]], axis=0) * m * \
        jnp.concatenate([m[1:], z1], axis=0)
    z4 = jnp.zeros((1, 4), jnp.float32)
    planes = []
    for d in range(3):
        xd = xp_ref[d]
        left = jnp.concatenate([z4, xd[:-1]], axis=0)
        right = jnp.concatenate([xd[1:], z4], axis=0)
        planes.append(jnp.concatenate([left, xd, right], axis=1))
    pad = jnp.zeros((n, TW - 37), jnp.float32)
    t_ref[...] = jnp.concatenate(planes + [m3, pad], axis=1)


def _build_table(xp, c2):
    return pl.pallas_call(
        _table_body,
        out_shape=jax.ShapeDtypeStruct((SEQ, TW), jnp.float32),
    )(xp, c2)


def _sc_gather(table, idx2d):
    bpw = NE // NW
    nch = bpw // CH
    mesh = plsc.VectorSubcoreMesh(core_axis_name="c", subcore_axis_name="s")

    @functools.partial(
        pl.kernel,
        mesh=mesh,
        out_type=jax.ShapeDtypeStruct((NE, TW), jnp.float32),
        scratch_types=[
            pltpu.VMEM((nch, CH), jnp.int32),
            pltpu.VMEM((4 * CH, TW), jnp.float32),
            pltpu.SemaphoreType.DMA,
        ],
        compiler_params=pltpu.CompilerParams(use_tc_tiling_on_sc=True),
    )
    def gk(table_hbm, idx_hbm, out_hbm, idx_v, rows_v, sem):
        wid = lax.axis_index("s") * NCORES + lax.axis_index("c")
        base = wid * bpw
        pltpu.sync_copy(idx_hbm.at[pl.ds(wid * nch, nch)], idx_v)
        for r in range(nch // 4):
            copies = []
            for jj in range(4):
                copies.append(pltpu.async_copy(
                    table_hbm.at[idx_v.at[r * 4 + jj]],
                    rows_v.at[pl.ds(jj * CH, CH)],
                    sem,
                ))
            for c in copies:
                c.wait()
            pltpu.sync_copy(
                rows_v, out_hbm.at[pl.ds(base + r * 4 * CH, 4 * CH)])

    return gk(table, idx2d)


BL = 16
EB = BL * K
CW = 640


def _diff_matrix():
    import numpy as np
    md = np.zeros((72, 3 * CW), np.float32)
    for d in range(3):
        for p in range(24):
            for q in range(24):
                col = CW * d + 24 * p + q
                md[24 * d + p, col] += 1.0
                md[24 * d + q, col] -= 1.0
    return jnp.asarray(md)


def _edge_body(t_ref, g_ref, md_ref, o_ref):
    ti = t_ref[...]
    g = g_ref[...]
    tib = jnp.broadcast_to(ti[:, None, :], (BL, K, TW)).reshape(EB, TW)
    mask = tib[:, 36:37] * g[:, 36:37]
    q72 = jnp.concatenate(
        [jnp.concatenate(
            [tib[:, 12 * d:12 * (d + 1)], g[:, 12 * d:12 * (d + 1)]], axis=1)
         for d in range(3)], axis=1)
    md = md_ref[...].astype(jnp.bfloat16)
    q_hi = q72.astype(jnp.bfloat16)
    q_lo = (q72 - q_hi.astype(jnp.float32)).astype(jnp.bfloat16)
    diff = (jnp.dot(q_hi, md, preferred_element_type=jnp.float32)
            + jnp.dot(q_lo, md, preferred_element_type=jnp.float32))
    sq = diff * diff
    ssq = sq[:, :CW] + sq[:, CW:2 * CW] + sq[:, 2 * CW:] + SQRT_EPS
    feat = jnp.log(jnp.sqrt(ssq) + DIST_EPS) * mask
    o_ref[...] = feat[:, :576]


def _edge_features(table, g):
    return pl.pallas_call(
        _edge_body,
        grid=(SEQ // BL,),
        in_specs=[
            pl.BlockSpec((BL, TW), lambda i: (i, 0)),
            pl.BlockSpec((EB, TW), lambda i: (i, 0)),
            pl.BlockSpec((72, 3 * CW), lambda i: (0, 0)),
        ],
        out_specs=pl.BlockSpec((EB, 576), lambda i: (i, 0)),
        out_shape=jax.ShapeDtypeStruct((NE, 576), jnp.float32),
    )(table, g, _diff_matrix())


def kernel(X, edge_idx, C):
    b, n = X.shape[0], X.shape[1]
    xp = jnp.transpose(X[0], (2, 0, 1))
    c2 = C[0][:, None].astype(jnp.int32)
    table = _build_table(xp, c2)
    idx2d = edge_idx[0].reshape(NE // CH, CH).astype(jnp.int32)
    g = _sc_gather(table, idx2d)
    out = _edge_features(table, g)
    return out.reshape(b, n, K, 576)
